# Initial kernel scaffold; baseline (speedup 1.0000x reference)
import jax
import jax.numpy as jnp
from jax.experimental import pallas as pl


def kernel(verts, vnormals, x, batch, edge_index, W1, b1, W2, b2, Win, Wc1, bc1, Wc2, Wout, bout):
    raise NotImplementedError("write your pallas kernel here")


# speedup vs baseline: 4.0467x; 4.0467x over previous
import functools

import jax
import jax.numpy as jnp
from jax import lax
from jax.experimental import pallas as pl
from jax.experimental.pallas import tpu as pltpu
from jax.experimental.pallas import tpu_sc as plsc

SCALES = (1.0, 2.0, 3.0, 5.0, 10.0)
RADIUS = 9.0
L = 16
NSUB = 16
NCORE = 2
EB = 80
RB = 1000
TW = 128


def _rsqrt(x):
    i = plsc.bitcast(x, jnp.int32)
    i = jnp.int32(0x5F3759DF) - lax.shift_right_logical(i, 1)
    y = plsc.bitcast(i, jnp.float32)
    for _ in range(3):
        y = y * (1.5 - 0.5 * x * y * y)
    return y


def _mesh():
    return plsc.VectorSubcoreMesh(
        core_axis_name="c", subcore_axis_name="s",
        num_cores=NCORE, num_subcores=NSUB)


_CP = dict(compiler_params=pltpu.CompilerParams(needs_layout_passes=False))


def _splat(v):
    return jnp.full((L,), v, jnp.int32)


def _stripes(n):
    NP0 = (n // NSUB) // 8 * 8
    NPL = n - (NSUB - 1) * NP0
    assert NPL % 8 == 0
    return NP0, NPL


WB = 4000


def _stripe_copy1d(s, n, src_at, dst_at, buf):
    NP0, NPL = _stripes(n)

    def do(off0, m):
        full, rem = divmod(m, WB)
        for k in range(full):
            o = off0 + k * WB
            pltpu.sync_copy(src_at(o, WB), buf)
            pltpu.sync_copy(buf, dst_at(o, WB))
        if rem:
            o = off0 + full * WB
            pltpu.sync_copy(src_at(o, rem), buf.at[pl.ds(0, rem)])
            pltpu.sync_copy(buf.at[pl.ds(0, rem)], dst_at(o, rem))

    @pl.when(s != NSUB - 1)
    def _():
        do(s * NP0, NP0)

    @pl.when(s == NSUB - 1)
    def _():
        do((NSUB - 1) * NP0, NPL)


@functools.lru_cache(maxsize=None)
def _make_phase_a(NV, NE):
    D = 10
    ET = NE // NSUB
    nchunk = ET // EB
    ngrp = EB // L
    AW = NV * D

    def body(src_hbm, dst_hbm, vtab_hbm, zeros_hbm,
             out_hbm, ex_hbm, ey_hbm, ez_hbm, ed2_hbm, end_hbm,
             sidx, didx, srows, drows, idxb, msgb,
             ebx, eby, ebz, ebd2, ebnd, wbuf, acc, sem_s, sem_d):
        c = lax.axis_index("c")
        s = lax.axis_index("s")
        _stripe_copy1d(s, AW, lambda o, n: zeros_hbm.at[pl.ds(o, n)],
                       lambda o, n: acc.at[pl.ds(o, n)], wbuf)
        plsc.subcore_barrier()
        iota = lax.iota(jnp.int32, L)
        c_is0 = jnp.broadcast_to(c == 0, (L,))

        def chunk(i, carry):
            base = s * ET + i * EB
            esl = pl.ds(base, EB)
            pltpu.sync_copy(src_hbm.at[esl], sidx)
            pltpu.sync_copy(dst_hbm.at[esl], didx)
            cp_s = pltpu.async_copy(vtab_hbm.at[sidx], srows, sem_s)
            cp_d = pltpu.async_copy(vtab_hbm.at[didx], drows, sem_d)
            cp_s.wait()
            cp_d.wait()

            def grp(g, carry2):
                rid = g * L + iota
                sl = pl.ds(g * L, L)

                def scol(ci):
                    return plsc.load_gather(srows, [rid, _splat(ci)])

                def dcol(ci):
                    return plsc.load_gather(drows, [rid, _splat(ci)])

                dxx = scol(0) - dcol(0)
                dxy = scol(1) - dcol(1)
                dxz = scol(2) - dcol(2)
                ndx = dcol(3)
                ndy = dcol(4)
                ndz = dcol(5)
                d2 = dxx * dxx + dxy * dxy + dxz * dxz
                d2e = d2 + 1e-12
                d1 = d2e * _rsqrt(d2e)
                dndx = ((scol(3) - ndx) * dxx + (scol(4) - ndy) * dxy
                        + (scol(5) - ndz) * dxz)
                nddx = ndx * dxx + ndy * dxy + ndz * dxz
                ch = []
                for sc in SCALES:
                    wgt = jnp.exp(d2 * (-0.5 / (sc * sc)))
                    ch += [wgt * d2, wgt * d1, wgt * dndx, wgt * nddx]
                di = didx[sl] * D
                for t in range(D):
                    msgb[t][sl] = jnp.where(c_is0, ch[t], ch[D + t])
                    idxb[t][sl] = di + t
                ebx[sl] = dxx
                eby[sl] = dxy
                ebz[sl] = dxz
                ebd2[sl] = d2
                ebnd[sl] = nddx
                return carry2

            lax.fori_loop(0, ngrp, grp, 0)
            for t in range(D):
                pltpu.sync_copy(msgb[t], acc.at[idxb[t]], add=True)

            @pl.when(c == 0)
            def _():
                pltpu.sync_copy(ebx, ex_hbm.at[esl])
                pltpu.sync_copy(eby, ey_hbm.at[esl])
                pltpu.sync_copy(ebz, ez_hbm.at[esl])
                pltpu.sync_copy(ebd2, ed2_hbm.at[esl])
                pltpu.sync_copy(ebnd, end_hbm.at[esl])
            return carry

        lax.fori_loop(0, nchunk, chunk, 0)
        plsc.subcore_barrier()
        _stripe_copy1d(s, AW, lambda o, n: acc.at[pl.ds(o, n)],
                       lambda o, n: out_hbm.at[pl.ds(c * AW + o, n)], wbuf)

    return pl.kernel(
        body,
        out_type=(jax.ShapeDtypeStruct((NCORE * AW,), jnp.float32),
                  jax.ShapeDtypeStruct((NE,), jnp.float32),
                  jax.ShapeDtypeStruct((NE,), jnp.float32),
                  jax.ShapeDtypeStruct((NE,), jnp.float32),
                  jax.ShapeDtypeStruct((NE,), jnp.float32),
                  jax.ShapeDtypeStruct((NE,), jnp.float32)),
        mesh=_mesh(), **_CP,
        scratch_types=[
            pltpu.VMEM((EB,), jnp.int32),
            pltpu.VMEM((EB,), jnp.int32),
            pltpu.VMEM((EB, TW), jnp.float32),
            pltpu.VMEM((EB, TW), jnp.float32),
            [pltpu.VMEM((EB,), jnp.int32)] * 10,
            [pltpu.VMEM((EB,), jnp.float32)] * 10,
            pltpu.VMEM((EB,), jnp.float32),
            pltpu.VMEM((EB,), jnp.float32),
            pltpu.VMEM((EB,), jnp.float32),
            pltpu.VMEM((EB,), jnp.float32),
            pltpu.VMEM((EB,), jnp.float32),
            pltpu.VMEM((WB,), jnp.float32),
            pltpu.VMEM_SHARED((NV * 10,), jnp.float32),
            pltpu.SemaphoreType.DMA,
            pltpu.SemaphoreType.DMA,
        ])


@functools.lru_cache(maxsize=None)
def _make_phase_b(NV, NE):
    D = 3
    EW = NE // (NCORE * NSUB)
    nchunk = EW // EB
    ngrp = EB // L
    AW = NV * D

    def body(src_hbm, dst_hbm, vtab_hbm, wvec_hbm, zeros_hbm,
             ex_hbm, ey_hbm, ez_hbm, end_hbm, out_hbm,
             sidx, didx, drows, swv, idxb, msgb, ebx, eby, ebz, ebnd,
             wbuf, acc, sem_d, sem_w):
        c = lax.axis_index("c")
        s = lax.axis_index("s")
        w = c * NSUB + s
        _stripe_copy1d(s, AW, lambda o, n: zeros_hbm.at[pl.ds(o, n)],
                       lambda o, n: acc.at[pl.ds(o, n)], wbuf)
        plsc.subcore_barrier()
        iota = lax.iota(jnp.int32, L)

        def chunk(i, carry):
            base = w * EW + i * EB
            esl = pl.ds(base, EB)
            pltpu.sync_copy(src_hbm.at[esl], sidx)
            pltpu.sync_copy(dst_hbm.at[esl], didx)
            pltpu.sync_copy(ex_hbm.at[esl], ebx)
            pltpu.sync_copy(ey_hbm.at[esl], eby)
            pltpu.sync_copy(ez_hbm.at[esl], ebz)
            pltpu.sync_copy(end_hbm.at[esl], ebnd)
            cp_w = pltpu.async_copy(wvec_hbm.at[sidx], swv, sem_w)
            cp_d = pltpu.async_copy(vtab_hbm.at[didx], drows, sem_d)
            cp_w.wait()
            cp_d.wait()

            def grp(g, carry2):
                rid = g * L + iota
                sl = pl.ds(g * L, L)

                def dcol(ci):
                    return plsc.load_gather(drows, [rid, _splat(ci)])

                sw = swv[sl]
                ndx = dcol(3)
                ndy = dcol(4)
                ndz = dcol(5)
                dxx = ebx[sl]
                dxy = eby[sl]
                dxz = ebz[sl]
                nddx = ebnd[sl]
                di = didx[sl] * D
                msgb[0][sl] = sw * (dxx - ndx * nddx)
                msgb[1][sl] = sw * (dxy - ndy * nddx)
                msgb[2][sl] = sw * (dxz - ndz * nddx)
                for t in range(D):
                    idxb[t][sl] = di + t
                return carry2

            lax.fori_loop(0, ngrp, grp, 0)
            for t in range(D):
                pltpu.sync_copy(msgb[t], acc.at[idxb[t]], add=True)
            return carry

        lax.fori_loop(0, nchunk, chunk, 0)
        plsc.subcore_barrier()
        _stripe_copy1d(s, AW, lambda o, n: acc.at[pl.ds(o, n)],
                       lambda o, n: out_hbm.at[pl.ds(c * AW + o, n)], wbuf)

    return pl.kernel(
        body,
        out_type=jax.ShapeDtypeStruct((NCORE * AW,), jnp.float32),
        mesh=_mesh(), **_CP,
        scratch_types=[
            pltpu.VMEM((EB,), jnp.int32),
            pltpu.VMEM((EB,), jnp.int32),
            pltpu.VMEM((EB, TW), jnp.float32),
            pltpu.VMEM((EB,), jnp.float32),
            [pltpu.VMEM((EB,), jnp.int32)] * 3,
            [pltpu.VMEM((EB,), jnp.float32)] * 3,
            pltpu.VMEM((EB,), jnp.float32),
            pltpu.VMEM((EB,), jnp.float32),
            pltpu.VMEM((EB,), jnp.float32),
            pltpu.VMEM((EB,), jnp.float32),
            pltpu.VMEM((WB,), jnp.float32),
            pltpu.VMEM_SHARED((NV * 3,), jnp.float32),
            pltpu.SemaphoreType.DMA,
            pltpu.SemaphoreType.DMA,
        ])


@functools.lru_cache(maxsize=None)
def _make_phase_c1(NV, NE):
    EW = NE // (NCORE * NSUB)
    nchunk = EW // EB
    ngrp = EB // L

    def body(dst_hbm, dtab_hbm, ex_hbm, ey_hbm, ez_hbm,
             ecx_hbm, ecy_hbm,
             didx, drows, ebx, eby, ebz, ocx, ocy, sem_d):
        c = lax.axis_index("c")
        s = lax.axis_index("s")
        w = c * NSUB + s
        iota = lax.iota(jnp.int32, L)
        inv_r = 1.0 / RADIUS

        def chunk(i, carry):
            base = w * EW + i * EB
            esl = pl.ds(base, EB)
            pltpu.sync_copy(dst_hbm.at[esl], didx)
            pltpu.sync_copy(ex_hbm.at[esl], ebx)
            pltpu.sync_copy(ey_hbm.at[esl], eby)
            pltpu.sync_copy(ez_hbm.at[esl], ebz)
            pltpu.async_copy(dtab_hbm.at[didx], drows, sem_d).wait()

            def grp(g, carry2):
                rid = g * L + iota
                sl = pl.ds(g * L, L)

                def dcol(ci):
                    return plsc.load_gather(drows, [rid, _splat(ci)])

                dxx = ebx[sl]
                dxy = eby[sl]
                dxz = ebz[sl]
                ocx[sl] = (dxx * dcol(0) + dxy * dcol(1)
                           + dxz * dcol(2)) * inv_r
                ocy[sl] = (dxx * dcol(3) + dxy * dcol(4)
                           + dxz * dcol(5)) * inv_r
                return carry2

            lax.fori_loop(0, ngrp, grp, 0)
            pltpu.sync_copy(ocx, ecx_hbm.at[esl])
            pltpu.sync_copy(ocy, ecy_hbm.at[esl])
            return carry

        lax.fori_loop(0, nchunk, chunk, 0)

    return pl.kernel(
        body,
        out_type=(jax.ShapeDtypeStruct((NE,), jnp.float32),
                  jax.ShapeDtypeStruct((NE,), jnp.float32)),
        mesh=_mesh(), **_CP,
        scratch_types=[
            pltpu.VMEM((EB,), jnp.int32),
            pltpu.VMEM((EB, TW), jnp.float32),
            pltpu.VMEM((EB,), jnp.float32),
            pltpu.VMEM((EB,), jnp.float32),
            pltpu.VMEM((EB,), jnp.float32),
            pltpu.VMEM((EB,), jnp.float32),
            pltpu.VMEM((EB,), jnp.float32),
            pltpu.SemaphoreType.DMA,
        ])


@functools.lru_cache(maxsize=None)
def _make_phase_c2(NV, NE):
    D = 8
    ET = NE // NSUB
    nchunk = ET // EB
    ngrp = EB // L
    AW = NV * D

    def body(src_hbm, dst_hbm, h1d_hbm, ecx_hbm, ecy_hbm, end_hbm,
             ed2_hbm, wc1_hbm, bc1_hbm, wc2a_hbm, wc2b_hbm, zeros_hbm,
             out_hbm,
             sidx, didx, hidxb, hvals, idxb, msgb, ebcx, ebcy, ebnd, ebd2,
             wc1v, bc1v, wc2v, wbuf, acc, sem_h):
        c = lax.axis_index("c")
        s = lax.axis_index("s")
        _stripe_copy1d(s, AW, lambda o, n: zeros_hbm.at[pl.ds(o, n)],
                       lambda o, n: acc.at[pl.ds(o, n)], wbuf)
        pltpu.sync_copy(wc1_hbm, wc1v)
        pltpu.sync_copy(bc1_hbm, bc1v)

        @pl.when(c == 0)
        def _():
            pltpu.sync_copy(wc2a_hbm, wc2v)

        @pl.when(c == 1)
        def _():
            pltpu.sync_copy(wc2b_hbm, wc2v)

        plsc.subcore_barrier()
        iota = lax.iota(jnp.int32, L)
        inv_r = 1.0 / RADIUS
        inv_r2 = 1.0 / (RADIUS * RADIUS)
        jb = c * D

        def chunk(i, carry):
            base = s * ET + i * EB
            esl = pl.ds(base, EB)
            pltpu.sync_copy(src_hbm.at[esl], sidx)
            pltpu.sync_copy(dst_hbm.at[esl], didx)
            pltpu.sync_copy(ecx_hbm.at[esl], ebcx)
            pltpu.sync_copy(ecy_hbm.at[esl], ebcy)
            pltpu.sync_copy(end_hbm.at[esl], ebnd)
            pltpu.sync_copy(ed2_hbm.at[esl], ebd2)

            def gidx(g, carry2):
                sl = pl.ds(g * L, L)
                si16 = sidx[sl] * 16 + jb
                for j in range(D):
                    hidxb[j][sl] = si16 + j
                return carry2

            lax.fori_loop(0, ngrp, gidx, 0)
            cps = [pltpu.async_copy(h1d_hbm.at[hidxb[j]], hvals[j], sem_h)
                   for j in range(D)]
            for cp in cps:
                cp.wait()

            def grp(g, carry2):
                sl = pl.ds(g * L, L)
                cx = ebcx[sl]
                cy = ebcy[sl]
                cz = ebnd[sl] * inv_r
                d2 = ebd2[sl]
                win = jnp.maximum(1.0 - d2 * inv_r2, 0.0)
                fh = []
                for k in range(16):
                    a = (cx * wc1v[pl.ds(k * L, L)]
                         + cy * wc1v[pl.ds((16 + k) * L, L)]
                         + cz * wc1v[pl.ds((32 + k) * L, L)]
                         + bc1v[pl.ds(k * L, L)])
                    fh.append(jnp.maximum(a, 0.0))
                di = didx[sl] * D
                for j in range(D):
                    filt = fh[0] * wc2v[pl.ds(j * 16 * L, L)]
                    for k in range(1, 16):
                        filt = filt + fh[k] * wc2v[pl.ds((j * 16 + k) * L, L)]
                    msgb[j][sl] = win * filt * hvals[j][sl]
                    idxb[j][sl] = di + j
                return carry2

            lax.fori_loop(0, ngrp, grp, 0)
            for j in range(D):
                pltpu.sync_copy(msgb[j], acc.at[idxb[j]], add=True)
            return carry

        lax.fori_loop(0, nchunk, chunk, 0)
        plsc.subcore_barrier()
        _stripe_copy1d(s, AW, lambda o, n: acc.at[pl.ds(o, n)],
                       lambda o, n: out_hbm.at[pl.ds(c * AW + o, n)], wbuf)

    return pl.kernel(
        body,
        out_type=jax.ShapeDtypeStruct((NCORE * AW,), jnp.float32),
        mesh=_mesh(), **_CP,
        scratch_types=[
            pltpu.VMEM((EB,), jnp.int32),
            pltpu.VMEM((EB,), jnp.int32),
            [pltpu.VMEM((EB,), jnp.int32)] * 8,
            [pltpu.VMEM((EB,), jnp.float32)] * 8,
            [pltpu.VMEM((EB,), jnp.int32)] * 8,
            [pltpu.VMEM((EB,), jnp.float32)] * 8,
            pltpu.VMEM((EB,), jnp.float32),
            pltpu.VMEM((EB,), jnp.float32),
            pltpu.VMEM((EB,), jnp.float32),
            pltpu.VMEM((EB,), jnp.float32),
            pltpu.VMEM((3 * 16 * L,), jnp.float32),
            pltpu.VMEM((16 * L,), jnp.float32),
            pltpu.VMEM((8 * 16 * L,), jnp.float32),
            pltpu.VMEM((WB,), jnp.float32),
            pltpu.VMEM_SHARED((NV * 8,), jnp.float32),
            pltpu.SemaphoreType.DMA,
        ])


def _row_spec(cols):
    return pl.BlockSpec((RB, cols), lambda i: (i, 0))


def _full_spec(shape):
    return pl.BlockSpec(shape, lambda i: tuple(0 for _ in shape))


def _t0_body(v_ref, vn_ref, out_ref):
    v = v_ref[...]
    vn = vn_ref[...]
    nrm = jnp.sqrt(jnp.sum(vn * vn, axis=1, keepdims=True))
    n = vn / (nrm + 1e-8)
    out_ref[...] = jnp.concatenate(
        [v, n, jnp.zeros((v.shape[0], TW - 6), jnp.float32)], axis=1)


@functools.lru_cache(maxsize=None)
def _make_t0(NV):
    return pl.pallas_call(
        _t0_body,
        grid=(NV // RB,),
        in_specs=[_row_spec(3), _row_spec(3)],
        out_specs=_row_spec(TW),
        out_shape=jax.ShapeDtypeStruct((NV, TW), jnp.float32),
    )


def _t2_body(a0_ref, a1_ref, x_ref, vtab_ref, w1_ref, b1_ref, w2_ref, b2_ref,
             win_ref, wvec_ref, h_ref, uv_ref):
    a = jnp.concatenate([a0_ref[...], a1_ref[...]], axis=1)
    R = a.shape[0]
    cols = []
    for k in range(len(SCALES)):
        denom2 = a[:, 4 * k + 0:4 * k + 1] + 1e-8
        denom1 = a[:, 4 * k + 1:4 * k + 2] + 1e-8
        cols.append(a[:, 4 * k + 2:4 * k + 3] / denom2)
        cols.append(a[:, 4 * k + 3:4 * k + 4] / denom1)
    x = x_ref[...]
    xf = jnp.concatenate([x] + cols + [jnp.zeros((R, 6), jnp.float32)], axis=1)
    hidd = jnp.dot(xf, w1_ref[...], preferred_element_type=jnp.float32)
    hidd = hidd + b1_ref[...]
    hidd = jnp.where(hidd >= 0, hidd, 0.2 * hidd)
    wv = jnp.dot(hidd, w2_ref[...], preferred_element_type=jnp.float32)
    wgt = wv[:, 0:1] + b2_ref[0, 0]
    h = jnp.maximum(
        jnp.dot(xf, win_ref[...], preferred_element_type=jnp.float32), 0.0)
    vtab = vtab_ref[...]
    nx = vtab[:, 3:4]
    ny = vtab[:, 4:5]
    nz = vtab[:, 5:6]
    zero = jnp.zeros((R, 1), jnp.float32)
    u0x, u0y, u0z = zero, nz, -ny
    nu0 = jnp.sqrt(u0y * u0y + u0z * u0z)
    pick = nu0 < 1e-4
    u0x = jnp.where(pick, -nz, u0x)
    u0y = jnp.where(pick, zero, u0y)
    u0z = jnp.where(pick, nx, u0z)
    inv = 1.0 / (jnp.sqrt(u0x * u0x + u0y * u0y + u0z * u0z) + 1e-8)
    u0x, u0y, u0z = u0x * inv, u0y * inv, u0z * inv
    v0x = ny * u0z - nz * u0y
    v0y = nz * u0x - nx * u0z
    v0z = nx * u0y - ny * u0x
    wvec_ref[...] = wgt
    h_ref[...] = h
    uv_ref[...] = jnp.concatenate(
        [u0x, u0y, u0z, v0x, v0y, v0z, zero, zero], axis=1)


@functools.lru_cache(maxsize=None)
def _make_t2(NV):
    return pl.pallas_call(
        _t2_body,
        grid=(NV // RB,),
        in_specs=[_row_spec(10), _row_spec(10), _row_spec(16), _row_spec(TW),
                  _full_spec((32, 16)), _full_spec((1, 16)),
                  _full_spec((16, 16)), _full_spec((1, 1)),
                  _full_spec((32, 16))],
        out_specs=[_row_spec(1), _row_spec(16), _row_spec(8)],
        out_shape=[jax.ShapeDtypeStruct((NV, 1), jnp.float32),
                   jax.ShapeDtypeStruct((NV, 16), jnp.float32),
                   jax.ShapeDtypeStruct((NV, 8), jnp.float32)],
    )


def _t4_body(t_ref, uv_ref, out_ref):
    t = t_ref[...]
    uv = uv_ref[...]
    tx, ty, tz = t[:, 0:1], t[:, 1:2], t[:, 2:3]
    u0x, u0y, u0z = uv[:, 0:1], uv[:, 1:2], uv[:, 2:3]
    v0x, v0y, v0z = uv[:, 3:4], uv[:, 4:5], uv[:, 5:6]
    tu = tx * u0x + ty * u0y + tz * u0z
    tv = tx * v0x + ty * v0y + tz * v0z
    tn = jnp.sqrt(tu * tu + tv * tv) + 1e-8
    co = tu / tn
    si = tv / tn
    ux, uy, uz = co * u0x + si * v0x, co * u0y + si * v0y, co * u0z + si * v0z
    vx, vy, vz = (co * v0x - si * u0x, co * v0y - si * u0y,
                  co * v0z - si * u0z)
    out_ref[...] = jnp.concatenate(
        [ux, uy, uz, vx, vy, vz,
         jnp.zeros((t.shape[0], TW - 6), jnp.float32)], axis=1)


@functools.lru_cache(maxsize=None)
def _make_t4(NV):
    return pl.pallas_call(
        _t4_body,
        grid=(NV // RB,),
        in_specs=[_row_spec(3), _row_spec(8)],
        out_specs=_row_spec(TW),
        out_shape=jax.ShapeDtypeStruct((NV, TW), jnp.float32),
    )


def _t6_body(a0_ref, a1_ref, wout_ref, bout_ref, out_ref):
    agg = jnp.concatenate([a0_ref[...], a1_ref[...]], axis=1)
    out_ref[...] = jnp.dot(agg, wout_ref[...],
                           preferred_element_type=jnp.float32) + bout_ref[...]


@functools.lru_cache(maxsize=None)
def _make_t6(NV):
    return pl.pallas_call(
        _t6_body,
        grid=(NV // RB,),
        in_specs=[_row_spec(8), _row_spec(8), _full_spec((16, 16)),
                  _full_spec((1, 16))],
        out_specs=_row_spec(16),
        out_shape=jax.ShapeDtypeStruct((NV, 16), jnp.float32),
    )


def kernel(verts, vnormals, x, batch, edge_index, W1, b1, W2, b2, Win, Wc1,
           bc1, Wc2, Wout, bout):
    NV = verts.shape[0]
    NE = edge_index.shape[1]
    src = edge_index[0]
    dst = edge_index[1]

    vtab = _make_t0(NV)(verts, vnormals)
    zeros10 = jnp.zeros((NV * 10,), jnp.float32)
    acc_a, ex, ey, ez, ed2, end_ = _make_phase_a(NV, NE)(
        src, dst, vtab, zeros10)
    a0 = acc_a[:NV * 10].reshape(NV, 10)
    a1 = acc_a[NV * 10:].reshape(NV, 10)

    W1p = jnp.zeros((32, 16), jnp.float32).at[:26].set(W1)
    Winp = jnp.zeros((32, 16), jnp.float32).at[:26].set(Win)
    W2p = jnp.zeros((16, 16), jnp.float32).at[:, 0:1].set(W2)
    wvec, h, uv = _make_t2(NV)(
        a0, a1, x, vtab, W1p, b1.reshape(1, 16), W2p, b2.reshape(1, 1), Winp)

    zeros3 = jnp.zeros((NV * 3,), jnp.float32)
    acc_b = _make_phase_b(NV, NE)(
        src, dst, vtab, wvec.reshape(NV), zeros3, ex, ey, ez, end_)
    t = (acc_b[:NV * 3] + acc_b[NV * 3:]).reshape(NV, 3)

    dtab = _make_t4(NV)(t, uv)
    ecx, ecy = _make_phase_c1(NV, NE)(dst, dtab, ex, ey, ez)

    wc1b = jnp.broadcast_to(
        Wc1[:, :, None], (3, 16, L)).astype(jnp.float32).reshape(3 * 16 * L)
    bc1b = jnp.broadcast_to(bc1[:, None], (16, L)).reshape(16 * L)
    wc2t = jnp.transpose(Wc2)
    wc2a = jnp.broadcast_to(wc2t[0:8][:, :, None], (8, 16, L)).reshape(-1)
    wc2b = jnp.broadcast_to(wc2t[8:16][:, :, None], (8, 16, L)).reshape(-1)
    zeros8 = jnp.zeros((NV * 8,), jnp.float32)
    acc_c = _make_phase_c2(NV, NE)(
        src, dst, h.reshape(NV * 16), ecx, ecy, end_, ed2, wc1b, bc1b,
        wc2a, wc2b, zeros8)
    g0 = acc_c[:NV * 8].reshape(NV, 8)
    g1 = acc_c[NV * 8:].reshape(NV, 8)

    return _make_t6(NV)(g0, g1, Wout, bout.reshape(1, 16))


# speedup vs baseline: 7.5037x; 1.8543x over previous
import functools

import jax
import jax.numpy as jnp
from jax import lax
from jax.experimental import pallas as pl
from jax.experimental.pallas import tpu as pltpu
from jax.experimental.pallas import tpu_sc as plsc

SCALES = (1.0, 2.0, 3.0, 5.0, 10.0)
RADIUS = 9.0
L = 16
NSUB = 16
NCORE = 2
EB = 400
EBC = 2000
RB = 1000
TW = 128


def _rsqrt(x):
    i = plsc.bitcast(x, jnp.int32)
    i = jnp.int32(0x5F3759DF) - lax.shift_right_logical(i, 1)
    y = plsc.bitcast(i, jnp.float32)
    for _ in range(3):
        y = y * (1.5 - 0.5 * x * y * y)
    return y


def _mesh():
    return plsc.VectorSubcoreMesh(
        core_axis_name="c", subcore_axis_name="s",
        num_cores=NCORE, num_subcores=NSUB)


_CP = dict(compiler_params=pltpu.CompilerParams(needs_layout_passes=False))


def _splat(v):
    return jnp.full((L,), v, jnp.int32)


def _stripes(n):
    NP0 = (n // NSUB) // 8 * 8
    NPL = n - (NSUB - 1) * NP0
    assert NPL % 8 == 0
    return NP0, NPL


WB = 4000


def _stripe_copy1d(s, n, src_at, dst_at, buf):
    NP0, NPL = _stripes(n)

    def do(off0, m):
        full, rem = divmod(m, WB)
        for k in range(full):
            o = off0 + k * WB
            pltpu.sync_copy(src_at(o, WB), buf)
            pltpu.sync_copy(buf, dst_at(o, WB))
        if rem:
            o = off0 + full * WB
            pltpu.sync_copy(src_at(o, rem), buf.at[pl.ds(0, rem)])
            pltpu.sync_copy(buf.at[pl.ds(0, rem)], dst_at(o, rem))

    @pl.when(s != NSUB - 1)
    def _():
        do(s * NP0, NP0)

    @pl.when(s == NSUB - 1)
    def _():
        do((NSUB - 1) * NP0, NPL)


@functools.lru_cache(maxsize=None)
def _make_phase_a(NV, NE):
    D = 10
    EB = 80
    ET = NE // NSUB
    nchunk = ET // EB
    ngrp = EB // L
    AW = NV * D

    def body(src_hbm, dst_hbm, vtab_hbm, zeros_hbm,
             out_hbm, ex_hbm, ey_hbm, ez_hbm, ed2_hbm, end_hbm,
             sidx, didx, srows, drows, idxb, msgb,
             ebx, eby, ebz, ebd2, ebnd, wbuf, acc, sem_s, sem_d):
        c = lax.axis_index("c")
        s = lax.axis_index("s")
        _stripe_copy1d(s, AW, lambda o, n: zeros_hbm.at[pl.ds(o, n)],
                       lambda o, n: acc.at[pl.ds(o, n)], wbuf)
        plsc.subcore_barrier()
        iota = lax.iota(jnp.int32, L)
        c_is0 = jnp.broadcast_to(c == 0, (L,))

        def chunk(i, carry):
            base = s * ET + i * EB
            esl = pl.ds(base, EB)
            pltpu.sync_copy(src_hbm.at[esl], sidx)
            pltpu.sync_copy(dst_hbm.at[esl], didx)
            cp_s = pltpu.async_copy(vtab_hbm.at[sidx], srows, sem_s)
            cp_d = pltpu.async_copy(vtab_hbm.at[didx], drows, sem_d)
            cp_s.wait()
            cp_d.wait()

            def grp(g, carry2):
                rid = g * L + iota
                sl = pl.ds(g * L, L)

                def scol(ci):
                    return plsc.load_gather(srows, [rid, _splat(ci)])

                def dcol(ci):
                    return plsc.load_gather(drows, [rid, _splat(ci)])

                dxx = scol(0) - dcol(0)
                dxy = scol(1) - dcol(1)
                dxz = scol(2) - dcol(2)
                ndx = dcol(3)
                ndy = dcol(4)
                ndz = dcol(5)
                d2 = dxx * dxx + dxy * dxy + dxz * dxz
                d2e = d2 + 1e-12
                d1 = d2e * _rsqrt(d2e)
                dndx = ((scol(3) - ndx) * dxx + (scol(4) - ndy) * dxy
                        + (scol(5) - ndz) * dxz)
                nddx = ndx * dxx + ndy * dxy + ndz * dxz
                ch = []
                for sc in SCALES:
                    wgt = jnp.exp(d2 * (-0.5 / (sc * sc)))
                    ch += [wgt * d2, wgt * d1, wgt * dndx, wgt * nddx]
                di = didx[sl] * D
                for t in range(D):
                    tsl = pl.ds(t * EB + g * L, L)
                    msgb[tsl] = jnp.where(c_is0, ch[t], ch[D + t])
                    idxb[tsl] = di + t
                ebx[sl] = dxx
                eby[sl] = dxy
                ebz[sl] = dxz
                ebd2[sl] = d2
                ebnd[sl] = nddx
                return carry2

            lax.fori_loop(0, ngrp, grp, 0)
            cp_a = pltpu.async_copy(msgb, acc.at[idxb], sem_s, add=True)

            @pl.when(c == 0)
            def _():
                ecs = [pltpu.async_copy(ebx, ex_hbm.at[esl], sem_d),
                       pltpu.async_copy(eby, ey_hbm.at[esl], sem_d),
                       pltpu.async_copy(ebz, ez_hbm.at[esl], sem_d),
                       pltpu.async_copy(ebd2, ed2_hbm.at[esl], sem_d),
                       pltpu.async_copy(ebnd, end_hbm.at[esl], sem_d)]
                for cp in ecs:
                    cp.wait()
            cp_a.wait()
            return carry

        lax.fori_loop(0, nchunk, chunk, 0)
        plsc.subcore_barrier()
        _stripe_copy1d(s, AW, lambda o, n: acc.at[pl.ds(o, n)],
                       lambda o, n: out_hbm.at[pl.ds(c * AW + o, n)], wbuf)

    return pl.kernel(
        body,
        out_type=(jax.ShapeDtypeStruct((NCORE * AW,), jnp.float32),
                  jax.ShapeDtypeStruct((NE,), jnp.float32),
                  jax.ShapeDtypeStruct((NE,), jnp.float32),
                  jax.ShapeDtypeStruct((NE,), jnp.float32),
                  jax.ShapeDtypeStruct((NE,), jnp.float32),
                  jax.ShapeDtypeStruct((NE,), jnp.float32)),
        mesh=_mesh(), **_CP,
        scratch_types=[
            pltpu.VMEM((EB,), jnp.int32),
            pltpu.VMEM((EB,), jnp.int32),
            pltpu.VMEM((EB, TW), jnp.float32),
            pltpu.VMEM((EB, TW), jnp.float32),
            pltpu.VMEM((EB * 10,), jnp.int32),
            pltpu.VMEM((EB * 10,), jnp.float32),
            pltpu.VMEM((EB,), jnp.float32),
            pltpu.VMEM((EB,), jnp.float32),
            pltpu.VMEM((EB,), jnp.float32),
            pltpu.VMEM((EB,), jnp.float32),
            pltpu.VMEM((EB,), jnp.float32),
            pltpu.VMEM((WB,), jnp.float32),
            pltpu.VMEM_SHARED((NV * 10,), jnp.float32),
            pltpu.SemaphoreType.DMA,
            pltpu.SemaphoreType.DMA,
        ])


@functools.lru_cache(maxsize=None)
def _make_phase_b(NV, NE):
    D = 3
    EW = NE // (NCORE * NSUB)
    nchunk = EW // EB
    ngrp = EB // L
    AW = NV * D

    def body(src_hbm, dst_hbm, vtab_hbm, wvec_hbm, zeros_hbm,
             ex_hbm, ey_hbm, ez_hbm, end_hbm, out_hbm,
             sidx, didx, drows, swv, idxb, msgb, ebx, eby, ebz, ebnd,
             wbuf, acc, sem_d, sem_w):
        c = lax.axis_index("c")
        s = lax.axis_index("s")
        w = c * NSUB + s
        _stripe_copy1d(s, AW, lambda o, n: zeros_hbm.at[pl.ds(o, n)],
                       lambda o, n: acc.at[pl.ds(o, n)], wbuf)
        plsc.subcore_barrier()
        iota = lax.iota(jnp.int32, L)

        def chunk(i, carry):
            base = w * EW + i * EB
            esl = pl.ds(base, EB)
            pltpu.sync_copy(src_hbm.at[esl], sidx)
            pltpu.sync_copy(dst_hbm.at[esl], didx)
            ecs = [pltpu.async_copy(ex_hbm.at[esl], ebx, sem_w),
                   pltpu.async_copy(ey_hbm.at[esl], eby, sem_w),
                   pltpu.async_copy(ez_hbm.at[esl], ebz, sem_w),
                   pltpu.async_copy(end_hbm.at[esl], ebnd, sem_w)]
            cp_w = pltpu.async_copy(wvec_hbm.at[sidx], swv, sem_w)
            cp_d = pltpu.async_copy(vtab_hbm.at[didx], drows, sem_d)
            for cp in ecs:
                cp.wait()
            cp_w.wait()
            cp_d.wait()

            def grp(g, carry2):
                rid = g * L + iota
                sl = pl.ds(g * L, L)

                def dcol(ci):
                    return plsc.load_gather(drows, [rid, _splat(ci)])

                sw = swv[sl]
                ndx = dcol(3)
                ndy = dcol(4)
                ndz = dcol(5)
                dxx = ebx[sl]
                dxy = eby[sl]
                dxz = ebz[sl]
                nddx = ebnd[sl]
                di = didx[sl] * D
                msgb[pl.ds(0 * EB + g * L, L)] = sw * (dxx - ndx * nddx)
                msgb[pl.ds(1 * EB + g * L, L)] = sw * (dxy - ndy * nddx)
                msgb[pl.ds(2 * EB + g * L, L)] = sw * (dxz - ndz * nddx)
                for t in range(D):
                    idxb[pl.ds(t * EB + g * L, L)] = di + t
                return carry2

            lax.fori_loop(0, ngrp, grp, 0)
            pltpu.async_copy(msgb, acc.at[idxb], sem_w, add=True).wait()
            return carry

        lax.fori_loop(0, nchunk, chunk, 0)
        plsc.subcore_barrier()
        _stripe_copy1d(s, AW, lambda o, n: acc.at[pl.ds(o, n)],
                       lambda o, n: out_hbm.at[pl.ds(c * AW + o, n)], wbuf)

    return pl.kernel(
        body,
        out_type=jax.ShapeDtypeStruct((NCORE * AW,), jnp.float32),
        mesh=_mesh(), **_CP,
        scratch_types=[
            pltpu.VMEM((EB,), jnp.int32),
            pltpu.VMEM((EB,), jnp.int32),
            pltpu.VMEM((EB, TW), jnp.float32),
            pltpu.VMEM((EB,), jnp.float32),
            pltpu.VMEM((EB * 3,), jnp.int32),
            pltpu.VMEM((EB * 3,), jnp.float32),
            pltpu.VMEM((EB,), jnp.float32),
            pltpu.VMEM((EB,), jnp.float32),
            pltpu.VMEM((EB,), jnp.float32),
            pltpu.VMEM((EB,), jnp.float32),
            pltpu.VMEM((WB,), jnp.float32),
            pltpu.VMEM_SHARED((NV * 3,), jnp.float32),
            pltpu.SemaphoreType.DMA,
            pltpu.SemaphoreType.DMA,
        ])


@functools.lru_cache(maxsize=None)
def _make_phase_c1(NV, NE):
    EW = NE // (NCORE * NSUB)
    nchunk = EW // EB
    ngrp = EB // L

    def body(dst_hbm, dtab_hbm, ex_hbm, ey_hbm, ez_hbm,
             ecx_hbm, ecy_hbm,
             didx, drows, ebx, eby, ebz, ocx, ocy, sem_d):
        c = lax.axis_index("c")
        s = lax.axis_index("s")
        w = c * NSUB + s
        iota = lax.iota(jnp.int32, L)
        inv_r = 1.0 / RADIUS

        def chunk(i, carry):
            base = w * EW + i * EB
            esl = pl.ds(base, EB)
            pltpu.sync_copy(dst_hbm.at[esl], didx)
            ecs = [pltpu.async_copy(ex_hbm.at[esl], ebx, sem_d),
                   pltpu.async_copy(ey_hbm.at[esl], eby, sem_d),
                   pltpu.async_copy(ez_hbm.at[esl], ebz, sem_d)]
            cp_d = pltpu.async_copy(dtab_hbm.at[didx], drows, sem_d)
            for cp in ecs:
                cp.wait()
            cp_d.wait()

            def grp(g, carry2):
                rid = g * L + iota
                sl = pl.ds(g * L, L)

                def dcol(ci):
                    return plsc.load_gather(drows, [rid, _splat(ci)])

                dxx = ebx[sl]
                dxy = eby[sl]
                dxz = ebz[sl]
                ocx[sl] = (dxx * dcol(0) + dxy * dcol(1)
                           + dxz * dcol(2)) * inv_r
                ocy[sl] = (dxx * dcol(3) + dxy * dcol(4)
                           + dxz * dcol(5)) * inv_r
                return carry2

            lax.fori_loop(0, ngrp, grp, 0)
            cp1 = pltpu.async_copy(ocx, ecx_hbm.at[esl], sem_d)
            cp2 = pltpu.async_copy(ocy, ecy_hbm.at[esl], sem_d)
            cp1.wait()
            cp2.wait()
            return carry

        lax.fori_loop(0, nchunk, chunk, 0)

    return pl.kernel(
        body,
        out_type=(jax.ShapeDtypeStruct((NE,), jnp.float32),
                  jax.ShapeDtypeStruct((NE,), jnp.float32)),
        mesh=_mesh(), **_CP,
        scratch_types=[
            pltpu.VMEM((EB,), jnp.int32),
            pltpu.VMEM((EB, TW), jnp.float32),
            pltpu.VMEM((EB,), jnp.float32),
            pltpu.VMEM((EB,), jnp.float32),
            pltpu.VMEM((EB,), jnp.float32),
            pltpu.VMEM((EB,), jnp.float32),
            pltpu.VMEM((EB,), jnp.float32),
            pltpu.SemaphoreType.DMA,
        ])


@functools.lru_cache(maxsize=None)
def _make_phase_c2(NV, NE):
    D = 8
    EBC = 400
    ET = NE // NSUB
    nchunk = ET // EBC
    ngrp = EBC // L
    AW = NV * D

    def body(src_hbm, dst_hbm, h1d_hbm, ecx_hbm, ecy_hbm, end_hbm,
             ed2_hbm, wc1_hbm, bc1_hbm, wc2a_hbm, wc2b_hbm, zeros_hbm,
             out_hbm,
             sidx, didx, hidxb, hvals, idxb, msgb, ebcx, ebcy, ebnd, ebd2,
             wc1v, bc1v, wc2v, wbuf, acc, sem_h):
        c = lax.axis_index("c")
        s = lax.axis_index("s")
        _stripe_copy1d(s, AW, lambda o, n: zeros_hbm.at[pl.ds(o, n)],
                       lambda o, n: acc.at[pl.ds(o, n)], wbuf)
        pltpu.sync_copy(wc1_hbm, wc1v)
        pltpu.sync_copy(bc1_hbm, bc1v)

        @pl.when(c == 0)
        def _():
            pltpu.sync_copy(wc2a_hbm, wc2v)

        @pl.when(c == 1)
        def _():
            pltpu.sync_copy(wc2b_hbm, wc2v)

        plsc.subcore_barrier()
        iota = lax.iota(jnp.int32, L)
        inv_r = 1.0 / RADIUS
        inv_r2 = 1.0 / (RADIUS * RADIUS)
        jb = c * D

        def chunk(i, carry):
            base = s * ET + i * EBC
            esl = pl.ds(base, EBC)
            pltpu.sync_copy(src_hbm.at[esl], sidx)
            pltpu.sync_copy(dst_hbm.at[esl], didx)
            ecs = [pltpu.async_copy(ecx_hbm.at[esl], ebcx, sem_h),
                   pltpu.async_copy(ecy_hbm.at[esl], ebcy, sem_h),
                   pltpu.async_copy(end_hbm.at[esl], ebnd, sem_h),
                   pltpu.async_copy(ed2_hbm.at[esl], ebd2, sem_h)]
            for cp in ecs:
                cp.wait()

            def gidx(g, carry2):
                sl = pl.ds(g * L, L)
                si16 = sidx[sl] * 16 + jb
                for j in range(D):
                    hidxb[j][sl] = si16 + j
                return carry2

            lax.fori_loop(0, ngrp, gidx, 0)
            cps = [pltpu.async_copy(h1d_hbm.at[hidxb[j]], hvals[j], sem_h)
                   for j in range(D)]
            for cp in cps:
                cp.wait()

            def grp(g, carry2):
                sl = pl.ds(g * L, L)
                cx = ebcx[sl]
                cy = ebcy[sl]
                cz = ebnd[sl] * inv_r
                d2 = ebd2[sl]
                win = jnp.maximum(1.0 - d2 * inv_r2, 0.0)
                fh = []
                for k in range(16):
                    a = (cx * wc1v[pl.ds(k * L, L)]
                         + cy * wc1v[pl.ds((16 + k) * L, L)]
                         + cz * wc1v[pl.ds((32 + k) * L, L)]
                         + bc1v[pl.ds(k * L, L)])
                    fh.append(jnp.maximum(a, 0.0))
                di = didx[sl] * D
                for j in range(D):
                    filt = fh[0] * wc2v[pl.ds(j * 16 * L, L)]
                    for k in range(1, 16):
                        filt = filt + fh[k] * wc2v[pl.ds((j * 16 + k) * L, L)]
                    jsl = pl.ds(j * EBC + g * L, L)
                    msgb[jsl] = win * filt * hvals[j][sl]
                    idxb[jsl] = di + j
                return carry2

            lax.fori_loop(0, ngrp, grp, 0)
            pltpu.async_copy(msgb, acc.at[idxb], sem_h, add=True).wait()
            return carry

        lax.fori_loop(0, nchunk, chunk, 0)
        plsc.subcore_barrier()
        _stripe_copy1d(s, AW, lambda o, n: acc.at[pl.ds(o, n)],
                       lambda o, n: out_hbm.at[pl.ds(c * AW + o, n)], wbuf)

    return pl.kernel(
        body,
        out_type=jax.ShapeDtypeStruct((NCORE * AW,), jnp.float32),
        mesh=_mesh(), **_CP,
        scratch_types=[
            pltpu.VMEM((EBC,), jnp.int32),
            pltpu.VMEM((EBC,), jnp.int32),
            [pltpu.VMEM((EBC,), jnp.int32)] * 8,
            [pltpu.VMEM((EBC,), jnp.float32)] * 8,
            pltpu.VMEM((EBC * 8,), jnp.int32),
            pltpu.VMEM((EBC * 8,), jnp.float32),
            pltpu.VMEM((EBC,), jnp.float32),
            pltpu.VMEM((EBC,), jnp.float32),
            pltpu.VMEM((EBC,), jnp.float32),
            pltpu.VMEM((EBC,), jnp.float32),
            pltpu.VMEM((3 * 16 * L,), jnp.float32),
            pltpu.VMEM((16 * L,), jnp.float32),
            pltpu.VMEM((8 * 16 * L,), jnp.float32),
            pltpu.VMEM((WB,), jnp.float32),
            pltpu.VMEM_SHARED((NV * 8,), jnp.float32),
            pltpu.SemaphoreType.DMA,
        ])


def _row_spec(cols):
    return pl.BlockSpec((RB, cols), lambda i: (i, 0))


def _full_spec(shape):
    return pl.BlockSpec(shape, lambda i: tuple(0 for _ in shape))


def _t0_body(v_ref, vn_ref, out_ref):
    v = v_ref[...]
    vn = vn_ref[...]
    nrm = jnp.sqrt(jnp.sum(vn * vn, axis=1, keepdims=True))
    n = vn / (nrm + 1e-8)
    out_ref[...] = jnp.concatenate(
        [v, n, jnp.zeros((v.shape[0], TW - 6), jnp.float32)], axis=1)


@functools.lru_cache(maxsize=None)
def _make_t0(NV):
    return pl.pallas_call(
        _t0_body,
        grid=(NV // RB,),
        in_specs=[_row_spec(3), _row_spec(3)],
        out_specs=_row_spec(TW),
        out_shape=jax.ShapeDtypeStruct((NV, TW), jnp.float32),
    )


def _t2_body(a0_ref, a1_ref, x_ref, vtab_ref, w1_ref, b1_ref, w2_ref, b2_ref,
             win_ref, wvec_ref, h_ref, uv_ref):
    a = jnp.concatenate([a0_ref[...], a1_ref[...]], axis=1)
    R = a.shape[0]
    cols = []
    for k in range(len(SCALES)):
        denom2 = a[:, 4 * k + 0:4 * k + 1] + 1e-8
        denom1 = a[:, 4 * k + 1:4 * k + 2] + 1e-8
        cols.append(a[:, 4 * k + 2:4 * k + 3] / denom2)
        cols.append(a[:, 4 * k + 3:4 * k + 4] / denom1)
    x = x_ref[...]
    xf = jnp.concatenate([x] + cols + [jnp.zeros((R, 6), jnp.float32)], axis=1)
    hidd = jnp.dot(xf, w1_ref[...], preferred_element_type=jnp.float32)
    hidd = hidd + b1_ref[...]
    hidd = jnp.where(hidd >= 0, hidd, 0.2 * hidd)
    wv = jnp.dot(hidd, w2_ref[...], preferred_element_type=jnp.float32)
    wgt = wv[:, 0:1] + b2_ref[0, 0]
    h = jnp.maximum(
        jnp.dot(xf, win_ref[...], preferred_element_type=jnp.float32), 0.0)
    vtab = vtab_ref[...]
    nx = vtab[:, 3:4]
    ny = vtab[:, 4:5]
    nz = vtab[:, 5:6]
    zero = jnp.zeros((R, 1), jnp.float32)
    u0x, u0y, u0z = zero, nz, -ny
    nu0 = jnp.sqrt(u0y * u0y + u0z * u0z)
    pick = nu0 < 1e-4
    u0x = jnp.where(pick, -nz, u0x)
    u0y = jnp.where(pick, zero, u0y)
    u0z = jnp.where(pick, nx, u0z)
    inv = 1.0 / (jnp.sqrt(u0x * u0x + u0y * u0y + u0z * u0z) + 1e-8)
    u0x, u0y, u0z = u0x * inv, u0y * inv, u0z * inv
    v0x = ny * u0z - nz * u0y
    v0y = nz * u0x - nx * u0z
    v0z = nx * u0y - ny * u0x
    wvec_ref[...] = wgt
    h_ref[...] = h
    uv_ref[...] = jnp.concatenate(
        [u0x, u0y, u0z, v0x, v0y, v0z, zero, zero], axis=1)


@functools.lru_cache(maxsize=None)
def _make_t2(NV):
    return pl.pallas_call(
        _t2_body,
        grid=(NV // RB,),
        in_specs=[_row_spec(10), _row_spec(10), _row_spec(16), _row_spec(TW),
                  _full_spec((32, 16)), _full_spec((1, 16)),
                  _full_spec((16, 16)), _full_spec((1, 1)),
                  _full_spec((32, 16))],
        out_specs=[_row_spec(1), _row_spec(16), _row_spec(8)],
        out_shape=[jax.ShapeDtypeStruct((NV, 1), jnp.float32),
                   jax.ShapeDtypeStruct((NV, 16), jnp.float32),
                   jax.ShapeDtypeStruct((NV, 8), jnp.float32)],
    )


def _t4_body(t_ref, uv_ref, out_ref):
    t = t_ref[...]
    uv = uv_ref[...]
    tx, ty, tz = t[:, 0:1], t[:, 1:2], t[:, 2:3]
    u0x, u0y, u0z = uv[:, 0:1], uv[:, 1:2], uv[:, 2:3]
    v0x, v0y, v0z = uv[:, 3:4], uv[:, 4:5], uv[:, 5:6]
    tu = tx * u0x + ty * u0y + tz * u0z
    tv = tx * v0x + ty * v0y + tz * v0z
    tn = jnp.sqrt(tu * tu + tv * tv) + 1e-8
    co = tu / tn
    si = tv / tn
    ux, uy, uz = co * u0x + si * v0x, co * u0y + si * v0y, co * u0z + si * v0z
    vx, vy, vz = (co * v0x - si * u0x, co * v0y - si * u0y,
                  co * v0z - si * u0z)
    out_ref[...] = jnp.concatenate(
        [ux, uy, uz, vx, vy, vz,
         jnp.zeros((t.shape[0], TW - 6), jnp.float32)], axis=1)


@functools.lru_cache(maxsize=None)
def _make_t4(NV):
    return pl.pallas_call(
        _t4_body,
        grid=(NV // RB,),
        in_specs=[_row_spec(3), _row_spec(8)],
        out_specs=_row_spec(TW),
        out_shape=jax.ShapeDtypeStruct((NV, TW), jnp.float32),
    )


def _t6_body(a0_ref, a1_ref, wout_ref, bout_ref, out_ref):
    agg = jnp.concatenate([a0_ref[...], a1_ref[...]], axis=1)
    out_ref[...] = jnp.dot(agg, wout_ref[...],
                           preferred_element_type=jnp.float32) + bout_ref[...]


@functools.lru_cache(maxsize=None)
def _make_t6(NV):
    return pl.pallas_call(
        _t6_body,
        grid=(NV // RB,),
        in_specs=[_row_spec(8), _row_spec(8), _full_spec((16, 16)),
                  _full_spec((1, 16))],
        out_specs=_row_spec(16),
        out_shape=jax.ShapeDtypeStruct((NV, 16), jnp.float32),
    )


def kernel(verts, vnormals, x, batch, edge_index, W1, b1, W2, b2, Win, Wc1,
           bc1, Wc2, Wout, bout):
    NV = verts.shape[0]
    NE = edge_index.shape[1]
    src = edge_index[0]
    dst = edge_index[1]

    vtab = _make_t0(NV)(verts, vnormals)
    zeros10 = jnp.zeros((NV * 10,), jnp.float32)
    acc_a, ex, ey, ez, ed2, end_ = _make_phase_a(NV, NE)(
        src, dst, vtab, zeros10)
    a0 = acc_a[:NV * 10].reshape(NV, 10)
    a1 = acc_a[NV * 10:].reshape(NV, 10)

    W1p = jnp.zeros((32, 16), jnp.float32).at[:26].set(W1)
    Winp = jnp.zeros((32, 16), jnp.float32).at[:26].set(Win)
    W2p = jnp.zeros((16, 16), jnp.float32).at[:, 0:1].set(W2)
    wvec, h, uv = _make_t2(NV)(
        a0, a1, x, vtab, W1p, b1.reshape(1, 16), W2p, b2.reshape(1, 1), Winp)

    zeros3 = jnp.zeros((NV * 3,), jnp.float32)
    acc_b = _make_phase_b(NV, NE)(
        src, dst, vtab, wvec.reshape(NV), zeros3, ex, ey, ez, end_)
    t = (acc_b[:NV * 3] + acc_b[NV * 3:]).reshape(NV, 3)

    dtab = _make_t4(NV)(t, uv)
    ecx, ecy = _make_phase_c1(NV, NE)(dst, dtab, ex, ey, ez)

    wc1b = jnp.broadcast_to(
        Wc1[:, :, None], (3, 16, L)).astype(jnp.float32).reshape(3 * 16 * L)
    bc1b = jnp.broadcast_to(bc1[:, None], (16, L)).reshape(16 * L)
    wc2t = jnp.transpose(Wc2)
    wc2a = jnp.broadcast_to(wc2t[0:8][:, :, None], (8, 16, L)).reshape(-1)
    wc2b = jnp.broadcast_to(wc2t[8:16][:, :, None], (8, 16, L)).reshape(-1)
    zeros8 = jnp.zeros((NV * 8,), jnp.float32)
    acc_c = _make_phase_c2(NV, NE)(
        src, dst, h.reshape(NV * 16), ecx, ecy, end_, ed2, wc1b, bc1b,
        wc2a, wc2b, zeros8)
    g0 = acc_c[:NV * 8].reshape(NV, 8)
    g1 = acc_c[NV * 8:].reshape(NV, 8)

    return _make_t6(NV)(g0, g1, Wout, bout.reshape(1, 16))


# speedup vs baseline: 7.5084x; 1.0006x over previous
import functools

import jax
import jax.numpy as jnp
from jax import lax
from jax.experimental import pallas as pl
from jax.experimental.pallas import tpu as pltpu
from jax.experimental.pallas import tpu_sc as plsc

SCALES = (1.0, 2.0, 3.0, 5.0, 10.0)
RADIUS = 9.0
L = 16
NSUB = 16
NCORE = 2
EB = 400
EBC = 2000
RB = 1000
TW = 128


def _rsqrt(x):
    i = plsc.bitcast(x, jnp.int32)
    i = jnp.int32(0x5F3759DF) - lax.shift_right_logical(i, 1)
    y = plsc.bitcast(i, jnp.float32)
    for _ in range(3):
        y = y * (1.5 - 0.5 * x * y * y)
    return y


def _mesh():
    return plsc.VectorSubcoreMesh(
        core_axis_name="c", subcore_axis_name="s",
        num_cores=NCORE, num_subcores=NSUB)


_CP = dict(compiler_params=pltpu.CompilerParams(needs_layout_passes=False))


def _splat(v):
    return jnp.full((L,), v, jnp.int32)


def _stripes(n):
    NP0 = (n // NSUB) // 8 * 8
    NPL = n - (NSUB - 1) * NP0
    assert NPL % 8 == 0
    return NP0, NPL


WB = 4000


def _stripe_copy1d(s, n, src_at, dst_at, buf):
    NP0, NPL = _stripes(n)

    def do(off0, m):
        full, rem = divmod(m, WB)
        for k in range(full):
            o = off0 + k * WB
            pltpu.sync_copy(src_at(o, WB), buf)
            pltpu.sync_copy(buf, dst_at(o, WB))
        if rem:
            o = off0 + full * WB
            pltpu.sync_copy(src_at(o, rem), buf.at[pl.ds(0, rem)])
            pltpu.sync_copy(buf.at[pl.ds(0, rem)], dst_at(o, rem))

    @pl.when(s != NSUB - 1)
    def _():
        do(s * NP0, NP0)

    @pl.when(s == NSUB - 1)
    def _():
        do((NSUB - 1) * NP0, NPL)


@functools.lru_cache(maxsize=None)
def _make_phase_a(NV, NE):
    D = 10
    EB = 80
    ET = NE // NSUB
    nchunk = ET // EB
    ngrp = EB // L
    AW = NV * D

    def body(src_hbm, dst_hbm, vtab_hbm, zeros_hbm,
             out_hbm, ex_hbm, ey_hbm, ez_hbm, ed2_hbm, end_hbm,
             sidx, didx, srows, drows, idxb, msgb,
             ebx, eby, ebz, ebd2, ebnd, wbuf, acc, sem_s, sem_d):
        c = lax.axis_index("c")
        s = lax.axis_index("s")
        _stripe_copy1d(s, AW, lambda o, n: zeros_hbm.at[pl.ds(o, n)],
                       lambda o, n: acc.at[pl.ds(o, n)], wbuf)
        plsc.subcore_barrier()
        iota = lax.iota(jnp.int32, L)
        c_is0 = jnp.broadcast_to(c == 0, (L,))

        def chunk(i, carry):
            base = s * ET + i * EB
            esl = pl.ds(base, EB)
            pltpu.sync_copy(src_hbm.at[esl], sidx)
            pltpu.sync_copy(dst_hbm.at[esl], didx)
            cp_s = pltpu.async_copy(vtab_hbm.at[sidx], srows, sem_s)
            cp_d = pltpu.async_copy(vtab_hbm.at[didx], drows, sem_d)
            cp_s.wait()
            cp_d.wait()

            def grp(g, carry2):
                rid = g * L + iota
                sl = pl.ds(g * L, L)

                def scol(ci):
                    return plsc.load_gather(srows, [rid, _splat(ci)])

                def dcol(ci):
                    return plsc.load_gather(drows, [rid, _splat(ci)])

                dxx = scol(0) - dcol(0)
                dxy = scol(1) - dcol(1)
                dxz = scol(2) - dcol(2)
                ndx = dcol(3)
                ndy = dcol(4)
                ndz = dcol(5)
                d2 = dxx * dxx + dxy * dxy + dxz * dxz
                d2e = d2 + 1e-12
                d1 = d2e * _rsqrt(d2e)
                dndx = ((scol(3) - ndx) * dxx + (scol(4) - ndy) * dxy
                        + (scol(5) - ndz) * dxz)
                nddx = ndx * dxx + ndy * dxy + ndz * dxz
                ch = []
                for sc in SCALES:
                    wgt = jnp.exp(d2 * (-0.5 / (sc * sc)))
                    ch += [wgt * d2, wgt * d1, wgt * dndx, wgt * nddx]
                di = didx[sl] * D
                for t in range(D):
                    tsl = pl.ds(t * EB + g * L, L)
                    msgb[tsl] = jnp.where(c_is0, ch[t], ch[D + t])
                    idxb[tsl] = di + t
                ebx[sl] = dxx
                eby[sl] = dxy
                ebz[sl] = dxz
                ebd2[sl] = d2
                ebnd[sl] = nddx
                return carry2

            lax.fori_loop(0, ngrp, grp, 0)
            cp_a = pltpu.async_copy(msgb, acc.at[idxb], sem_s, add=True)

            @pl.when(c == 0)
            def _():
                ecs = [pltpu.async_copy(ebx, ex_hbm.at[esl], sem_d),
                       pltpu.async_copy(eby, ey_hbm.at[esl], sem_d),
                       pltpu.async_copy(ebz, ez_hbm.at[esl], sem_d),
                       pltpu.async_copy(ebd2, ed2_hbm.at[esl], sem_d),
                       pltpu.async_copy(ebnd, end_hbm.at[esl], sem_d)]
                for cp in ecs:
                    cp.wait()
            cp_a.wait()
            return carry

        lax.fori_loop(0, nchunk, chunk, 0)
        plsc.subcore_barrier()
        _stripe_copy1d(s, AW, lambda o, n: acc.at[pl.ds(o, n)],
                       lambda o, n: out_hbm.at[pl.ds(c * AW + o, n)], wbuf)

    return pl.kernel(
        body,
        out_type=(jax.ShapeDtypeStruct((NCORE * AW,), jnp.float32),
                  jax.ShapeDtypeStruct((NE,), jnp.float32),
                  jax.ShapeDtypeStruct((NE,), jnp.float32),
                  jax.ShapeDtypeStruct((NE,), jnp.float32),
                  jax.ShapeDtypeStruct((NE,), jnp.float32),
                  jax.ShapeDtypeStruct((NE,), jnp.float32)),
        mesh=_mesh(), **_CP,
        scratch_types=[
            pltpu.VMEM((EB,), jnp.int32),
            pltpu.VMEM((EB,), jnp.int32),
            pltpu.VMEM((EB, TW), jnp.float32),
            pltpu.VMEM((EB, TW), jnp.float32),
            pltpu.VMEM((EB * 10,), jnp.int32),
            pltpu.VMEM((EB * 10,), jnp.float32),
            pltpu.VMEM((EB,), jnp.float32),
            pltpu.VMEM((EB,), jnp.float32),
            pltpu.VMEM((EB,), jnp.float32),
            pltpu.VMEM((EB,), jnp.float32),
            pltpu.VMEM((EB,), jnp.float32),
            pltpu.VMEM((WB,), jnp.float32),
            pltpu.VMEM_SHARED((NV * 10,), jnp.float32),
            pltpu.SemaphoreType.DMA,
            pltpu.SemaphoreType.DMA,
        ])


@functools.lru_cache(maxsize=None)
def _make_phase_b(NV, NE):
    D = 3
    EW = NE // (NCORE * NSUB)
    nchunk = EW // EB
    ngrp = EB // L
    AW = NV * D

    def body(src_hbm, dst_hbm, vtab_hbm, wvec_hbm, zeros_hbm,
             ex_hbm, ey_hbm, ez_hbm, end_hbm, out_hbm,
             sidx, didx, drows, swv, idxb, msgb, ebx, eby, ebz, ebnd,
             wbuf, acc, sem_d, sem_w):
        c = lax.axis_index("c")
        s = lax.axis_index("s")
        w = c * NSUB + s
        _stripe_copy1d(s, AW, lambda o, n: zeros_hbm.at[pl.ds(o, n)],
                       lambda o, n: acc.at[pl.ds(o, n)], wbuf)
        plsc.subcore_barrier()
        iota = lax.iota(jnp.int32, L)

        def chunk(i, carry):
            base = w * EW + i * EB
            esl = pl.ds(base, EB)
            pltpu.sync_copy(src_hbm.at[esl], sidx)
            pltpu.sync_copy(dst_hbm.at[esl], didx)
            ecs = [pltpu.async_copy(ex_hbm.at[esl], ebx, sem_w),
                   pltpu.async_copy(ey_hbm.at[esl], eby, sem_w),
                   pltpu.async_copy(ez_hbm.at[esl], ebz, sem_w),
                   pltpu.async_copy(end_hbm.at[esl], ebnd, sem_w)]
            cp_w = pltpu.async_copy(wvec_hbm.at[sidx], swv, sem_w)
            cp_d = pltpu.async_copy(vtab_hbm.at[didx], drows, sem_d)
            for cp in ecs:
                cp.wait()
            cp_w.wait()
            cp_d.wait()

            def grp(g, carry2):
                rid = g * L + iota
                sl = pl.ds(g * L, L)

                def dcol(ci):
                    return plsc.load_gather(drows, [rid, _splat(ci)])

                sw = swv[sl]
                ndx = dcol(3)
                ndy = dcol(4)
                ndz = dcol(5)
                dxx = ebx[sl]
                dxy = eby[sl]
                dxz = ebz[sl]
                nddx = ebnd[sl]
                di = didx[sl] * D
                msgb[pl.ds(0 * EB + g * L, L)] = sw * (dxx - ndx * nddx)
                msgb[pl.ds(1 * EB + g * L, L)] = sw * (dxy - ndy * nddx)
                msgb[pl.ds(2 * EB + g * L, L)] = sw * (dxz - ndz * nddx)
                for t in range(D):
                    idxb[pl.ds(t * EB + g * L, L)] = di + t
                return carry2

            lax.fori_loop(0, ngrp, grp, 0)
            pltpu.async_copy(msgb, acc.at[idxb], sem_w, add=True).wait()
            return carry

        lax.fori_loop(0, nchunk, chunk, 0)
        plsc.subcore_barrier()
        _stripe_copy1d(s, AW, lambda o, n: acc.at[pl.ds(o, n)],
                       lambda o, n: out_hbm.at[pl.ds(c * AW + o, n)], wbuf)

    return pl.kernel(
        body,
        out_type=jax.ShapeDtypeStruct((NCORE * AW,), jnp.float32),
        mesh=_mesh(), **_CP,
        scratch_types=[
            pltpu.VMEM((EB,), jnp.int32),
            pltpu.VMEM((EB,), jnp.int32),
            pltpu.VMEM((EB, TW), jnp.float32),
            pltpu.VMEM((EB,), jnp.float32),
            pltpu.VMEM((EB * 3,), jnp.int32),
            pltpu.VMEM((EB * 3,), jnp.float32),
            pltpu.VMEM((EB,), jnp.float32),
            pltpu.VMEM((EB,), jnp.float32),
            pltpu.VMEM((EB,), jnp.float32),
            pltpu.VMEM((EB,), jnp.float32),
            pltpu.VMEM((WB,), jnp.float32),
            pltpu.VMEM_SHARED((NV * 3,), jnp.float32),
            pltpu.SemaphoreType.DMA,
            pltpu.SemaphoreType.DMA,
        ])


@functools.lru_cache(maxsize=None)
def _make_phase_c1(NV, NE):
    EW = NE // (NCORE * NSUB)
    nchunk = EW // EB
    ngrp = EB // L

    def body(dst_hbm, dtab_hbm, ex_hbm, ey_hbm, ez_hbm,
             ecx_hbm, ecy_hbm,
             didx, drows, ebx, eby, ebz, ocx, ocy, sem_d):
        c = lax.axis_index("c")
        s = lax.axis_index("s")
        w = c * NSUB + s
        iota = lax.iota(jnp.int32, L)
        inv_r = 1.0 / RADIUS

        def chunk(i, carry):
            base = w * EW + i * EB
            esl = pl.ds(base, EB)
            pltpu.sync_copy(dst_hbm.at[esl], didx)
            ecs = [pltpu.async_copy(ex_hbm.at[esl], ebx, sem_d),
                   pltpu.async_copy(ey_hbm.at[esl], eby, sem_d),
                   pltpu.async_copy(ez_hbm.at[esl], ebz, sem_d)]
            cp_d = pltpu.async_copy(dtab_hbm.at[didx], drows, sem_d)
            for cp in ecs:
                cp.wait()
            cp_d.wait()

            def grp(g, carry2):
                rid = g * L + iota
                sl = pl.ds(g * L, L)

                def dcol(ci):
                    return plsc.load_gather(drows, [rid, _splat(ci)])

                dxx = ebx[sl]
                dxy = eby[sl]
                dxz = ebz[sl]
                ocx[sl] = (dxx * dcol(0) + dxy * dcol(1)
                           + dxz * dcol(2)) * inv_r
                ocy[sl] = (dxx * dcol(3) + dxy * dcol(4)
                           + dxz * dcol(5)) * inv_r
                return carry2

            lax.fori_loop(0, ngrp, grp, 0)
            cp1 = pltpu.async_copy(ocx, ecx_hbm.at[esl], sem_d)
            cp2 = pltpu.async_copy(ocy, ecy_hbm.at[esl], sem_d)
            cp1.wait()
            cp2.wait()
            return carry

        lax.fori_loop(0, nchunk, chunk, 0)

    return pl.kernel(
        body,
        out_type=(jax.ShapeDtypeStruct((NE,), jnp.float32),
                  jax.ShapeDtypeStruct((NE,), jnp.float32)),
        mesh=_mesh(), **_CP,
        scratch_types=[
            pltpu.VMEM((EB,), jnp.int32),
            pltpu.VMEM((EB, TW), jnp.float32),
            pltpu.VMEM((EB,), jnp.float32),
            pltpu.VMEM((EB,), jnp.float32),
            pltpu.VMEM((EB,), jnp.float32),
            pltpu.VMEM((EB,), jnp.float32),
            pltpu.VMEM((EB,), jnp.float32),
            pltpu.SemaphoreType.DMA,
        ])


@functools.lru_cache(maxsize=None)
def _make_phase_c2(NV, NE):
    D = 8
    EBC = 400
    ET = NE // NSUB
    nchunk = ET // EBC
    ngrp = EBC // L
    AW = NV * D

    def body(src_hbm, dst_hbm, htab_hbm, ecx_hbm, ecy_hbm, end_hbm,
             ed2_hbm, wc1_hbm, bc1_hbm, wc2a_hbm, wc2b_hbm, zeros_hbm,
             out_hbm,
             sidx, didx, srows, idxb, msgb, ebcx, ebcy, ebnd, ebd2,
             wc1v, bc1v, wc2v, wbuf, acc, sem_h):
        c = lax.axis_index("c")
        s = lax.axis_index("s")
        _stripe_copy1d(s, AW, lambda o, n: zeros_hbm.at[pl.ds(o, n)],
                       lambda o, n: acc.at[pl.ds(o, n)], wbuf)
        pltpu.sync_copy(wc1_hbm, wc1v)
        pltpu.sync_copy(bc1_hbm, bc1v)

        @pl.when(c == 0)
        def _():
            pltpu.sync_copy(wc2a_hbm, wc2v)

        @pl.when(c == 1)
        def _():
            pltpu.sync_copy(wc2b_hbm, wc2v)

        plsc.subcore_barrier()
        iota = lax.iota(jnp.int32, L)
        inv_r = 1.0 / RADIUS
        inv_r2 = 1.0 / (RADIUS * RADIUS)
        jb = c * D
        jcol = [jnp.broadcast_to(jb + j, (L,)) for j in range(D)]

        def chunk(i, carry):
            base = s * ET + i * EBC
            esl = pl.ds(base, EBC)
            pltpu.sync_copy(src_hbm.at[esl], sidx)
            pltpu.sync_copy(dst_hbm.at[esl], didx)
            ecs = [pltpu.async_copy(ecx_hbm.at[esl], ebcx, sem_h),
                   pltpu.async_copy(ecy_hbm.at[esl], ebcy, sem_h),
                   pltpu.async_copy(end_hbm.at[esl], ebnd, sem_h),
                   pltpu.async_copy(ed2_hbm.at[esl], ebd2, sem_h)]
            for cp in ecs:
                cp.wait()

            pltpu.async_copy(htab_hbm.at[sidx], srows, sem_h).wait()

            def grp(g, carry2):
                rid = g * L + iota
                sl = pl.ds(g * L, L)
                cx = ebcx[sl]
                cy = ebcy[sl]
                cz = ebnd[sl] * inv_r
                d2 = ebd2[sl]
                win = jnp.maximum(1.0 - d2 * inv_r2, 0.0)
                fh = []
                for k in range(16):
                    a = (cx * wc1v[pl.ds(k * L, L)]
                         + cy * wc1v[pl.ds((16 + k) * L, L)]
                         + cz * wc1v[pl.ds((32 + k) * L, L)]
                         + bc1v[pl.ds(k * L, L)])
                    fh.append(jnp.maximum(a, 0.0))
                di = didx[sl] * D
                for j in range(D):
                    filt = fh[0] * wc2v[pl.ds(j * 16 * L, L)]
                    for k in range(1, 16):
                        filt = filt + fh[k] * wc2v[pl.ds((j * 16 + k) * L, L)]
                    jsl = pl.ds(j * EBC + g * L, L)
                    hj = plsc.load_gather(srows, [rid, jcol[j]])
                    msgb[jsl] = win * filt * hj
                    idxb[jsl] = di + j
                return carry2

            lax.fori_loop(0, ngrp, grp, 0)
            pltpu.async_copy(msgb, acc.at[idxb], sem_h, add=True).wait()
            return carry

        lax.fori_loop(0, nchunk, chunk, 0)
        plsc.subcore_barrier()
        _stripe_copy1d(s, AW, lambda o, n: acc.at[pl.ds(o, n)],
                       lambda o, n: out_hbm.at[pl.ds(c * AW + o, n)], wbuf)

    return pl.kernel(
        body,
        out_type=jax.ShapeDtypeStruct((NCORE * AW,), jnp.float32),
        mesh=_mesh(), **_CP,
        scratch_types=[
            pltpu.VMEM((EBC,), jnp.int32),
            pltpu.VMEM((EBC,), jnp.int32),
            pltpu.VMEM((EBC, TW), jnp.float32),
            pltpu.VMEM((EBC * 8,), jnp.int32),
            pltpu.VMEM((EBC * 8,), jnp.float32),
            pltpu.VMEM((EBC,), jnp.float32),
            pltpu.VMEM((EBC,), jnp.float32),
            pltpu.VMEM((EBC,), jnp.float32),
            pltpu.VMEM((EBC,), jnp.float32),
            pltpu.VMEM((3 * 16 * L,), jnp.float32),
            pltpu.VMEM((16 * L,), jnp.float32),
            pltpu.VMEM((8 * 16 * L,), jnp.float32),
            pltpu.VMEM((WB,), jnp.float32),
            pltpu.VMEM_SHARED((NV * 8,), jnp.float32),
            pltpu.SemaphoreType.DMA,
        ])


def _row_spec(cols):
    return pl.BlockSpec((RB, cols), lambda i: (i, 0))


def _full_spec(shape):
    return pl.BlockSpec(shape, lambda i: tuple(0 for _ in shape))


def _t0_body(v_ref, vn_ref, out_ref):
    v = v_ref[...]
    vn = vn_ref[...]
    nrm = jnp.sqrt(jnp.sum(vn * vn, axis=1, keepdims=True))
    n = vn / (nrm + 1e-8)
    out_ref[...] = jnp.concatenate(
        [v, n, jnp.zeros((v.shape[0], TW - 6), jnp.float32)], axis=1)


@functools.lru_cache(maxsize=None)
def _make_t0(NV):
    return pl.pallas_call(
        _t0_body,
        grid=(NV // RB,),
        in_specs=[_row_spec(3), _row_spec(3)],
        out_specs=_row_spec(TW),
        out_shape=jax.ShapeDtypeStruct((NV, TW), jnp.float32),
    )


def _t2_body(a0_ref, a1_ref, x_ref, vtab_ref, w1_ref, b1_ref, w2_ref, b2_ref,
             win_ref, wvec_ref, h_ref, uv_ref):
    a = jnp.concatenate([a0_ref[...], a1_ref[...]], axis=1)
    R = a.shape[0]
    cols = []
    for k in range(len(SCALES)):
        denom2 = a[:, 4 * k + 0:4 * k + 1] + 1e-8
        denom1 = a[:, 4 * k + 1:4 * k + 2] + 1e-8
        cols.append(a[:, 4 * k + 2:4 * k + 3] / denom2)
        cols.append(a[:, 4 * k + 3:4 * k + 4] / denom1)
    x = x_ref[...]
    xf = jnp.concatenate([x] + cols + [jnp.zeros((R, 6), jnp.float32)], axis=1)
    hidd = jnp.dot(xf, w1_ref[...], preferred_element_type=jnp.float32)
    hidd = hidd + b1_ref[...]
    hidd = jnp.where(hidd >= 0, hidd, 0.2 * hidd)
    wv = jnp.dot(hidd, w2_ref[...], preferred_element_type=jnp.float32)
    wgt = wv[:, 0:1] + b2_ref[0, 0]
    h = jnp.maximum(
        jnp.dot(xf, win_ref[...], preferred_element_type=jnp.float32), 0.0)
    vtab = vtab_ref[...]
    nx = vtab[:, 3:4]
    ny = vtab[:, 4:5]
    nz = vtab[:, 5:6]
    zero = jnp.zeros((R, 1), jnp.float32)
    u0x, u0y, u0z = zero, nz, -ny
    nu0 = jnp.sqrt(u0y * u0y + u0z * u0z)
    pick = nu0 < 1e-4
    u0x = jnp.where(pick, -nz, u0x)
    u0y = jnp.where(pick, zero, u0y)
    u0z = jnp.where(pick, nx, u0z)
    inv = 1.0 / (jnp.sqrt(u0x * u0x + u0y * u0y + u0z * u0z) + 1e-8)
    u0x, u0y, u0z = u0x * inv, u0y * inv, u0z * inv
    v0x = ny * u0z - nz * u0y
    v0y = nz * u0x - nx * u0z
    v0z = nx * u0y - ny * u0x
    wvec_ref[...] = wgt
    h_ref[...] = jnp.concatenate(
        [h, jnp.zeros((R, TW - 16), jnp.float32)], axis=1)
    uv_ref[...] = jnp.concatenate(
        [u0x, u0y, u0z, v0x, v0y, v0z, zero, zero], axis=1)


@functools.lru_cache(maxsize=None)
def _make_t2(NV):
    return pl.pallas_call(
        _t2_body,
        grid=(NV // RB,),
        in_specs=[_row_spec(10), _row_spec(10), _row_spec(16), _row_spec(TW),
                  _full_spec((32, 16)), _full_spec((1, 16)),
                  _full_spec((16, 16)), _full_spec((1, 1)),
                  _full_spec((32, 16))],
        out_specs=[_row_spec(1), _row_spec(TW), _row_spec(8)],
        out_shape=[jax.ShapeDtypeStruct((NV, 1), jnp.float32),
                   jax.ShapeDtypeStruct((NV, TW), jnp.float32),
                   jax.ShapeDtypeStruct((NV, 8), jnp.float32)],
    )


def _t4_body(t_ref, uv_ref, out_ref):
    t = t_ref[...]
    uv = uv_ref[...]
    tx, ty, tz = t[:, 0:1], t[:, 1:2], t[:, 2:3]
    u0x, u0y, u0z = uv[:, 0:1], uv[:, 1:2], uv[:, 2:3]
    v0x, v0y, v0z = uv[:, 3:4], uv[:, 4:5], uv[:, 5:6]
    tu = tx * u0x + ty * u0y + tz * u0z
    tv = tx * v0x + ty * v0y + tz * v0z
    tn = jnp.sqrt(tu * tu + tv * tv) + 1e-8
    co = tu / tn
    si = tv / tn
    ux, uy, uz = co * u0x + si * v0x, co * u0y + si * v0y, co * u0z + si * v0z
    vx, vy, vz = (co * v0x - si * u0x, co * v0y - si * u0y,
                  co * v0z - si * u0z)
    out_ref[...] = jnp.concatenate(
        [ux, uy, uz, vx, vy, vz,
         jnp.zeros((t.shape[0], TW - 6), jnp.float32)], axis=1)


@functools.lru_cache(maxsize=None)
def _make_t4(NV):
    return pl.pallas_call(
        _t4_body,
        grid=(NV // RB,),
        in_specs=[_row_spec(3), _row_spec(8)],
        out_specs=_row_spec(TW),
        out_shape=jax.ShapeDtypeStruct((NV, TW), jnp.float32),
    )


def _t6_body(a0_ref, a1_ref, wout_ref, bout_ref, out_ref):
    agg = jnp.concatenate([a0_ref[...], a1_ref[...]], axis=1)
    out_ref[...] = jnp.dot(agg, wout_ref[...],
                           preferred_element_type=jnp.float32) + bout_ref[...]


@functools.lru_cache(maxsize=None)
def _make_t6(NV):
    return pl.pallas_call(
        _t6_body,
        grid=(NV // RB,),
        in_specs=[_row_spec(8), _row_spec(8), _full_spec((16, 16)),
                  _full_spec((1, 16))],
        out_specs=_row_spec(16),
        out_shape=jax.ShapeDtypeStruct((NV, 16), jnp.float32),
    )


def kernel(verts, vnormals, x, batch, edge_index, W1, b1, W2, b2, Win, Wc1,
           bc1, Wc2, Wout, bout):
    NV = verts.shape[0]
    NE = edge_index.shape[1]
    src = edge_index[0]
    dst = edge_index[1]

    vtab = _make_t0(NV)(verts, vnormals)
    zeros10 = jnp.zeros((NV * 10,), jnp.float32)
    acc_a, ex, ey, ez, ed2, end_ = _make_phase_a(NV, NE)(
        src, dst, vtab, zeros10)
    a0 = acc_a[:NV * 10].reshape(NV, 10)
    a1 = acc_a[NV * 10:].reshape(NV, 10)

    W1p = jnp.zeros((32, 16), jnp.float32).at[:26].set(W1)
    Winp = jnp.zeros((32, 16), jnp.float32).at[:26].set(Win)
    W2p = jnp.zeros((16, 16), jnp.float32).at[:, 0:1].set(W2)
    wvec, h, uv = _make_t2(NV)(
        a0, a1, x, vtab, W1p, b1.reshape(1, 16), W2p, b2.reshape(1, 1), Winp)

    zeros3 = jnp.zeros((NV * 3,), jnp.float32)
    acc_b = _make_phase_b(NV, NE)(
        src, dst, vtab, wvec.reshape(NV), zeros3, ex, ey, ez, end_)
    t = (acc_b[:NV * 3] + acc_b[NV * 3:]).reshape(NV, 3)

    dtab = _make_t4(NV)(t, uv)
    ecx, ecy = _make_phase_c1(NV, NE)(dst, dtab, ex, ey, ez)

    wc1b = jnp.broadcast_to(
        Wc1[:, :, None], (3, 16, L)).astype(jnp.float32).reshape(3 * 16 * L)
    bc1b = jnp.broadcast_to(bc1[:, None], (16, L)).reshape(16 * L)
    wc2t = jnp.transpose(Wc2)
    wc2a = jnp.broadcast_to(wc2t[0:8][:, :, None], (8, 16, L)).reshape(-1)
    wc2b = jnp.broadcast_to(wc2t[8:16][:, :, None], (8, 16, L)).reshape(-1)
    zeros8 = jnp.zeros((NV * 8,), jnp.float32)
    acc_c = _make_phase_c2(NV, NE)(
        src, dst, h, ecx, ecy, end_, ed2, wc1b, bc1b,
        wc2a, wc2b, zeros8)
    g0 = acc_c[:NV * 8].reshape(NV, 8)
    g1 = acc_c[NV * 8:].reshape(NV, 8)

    return _make_t6(NV)(g0, g1, Wout, bout.reshape(1, 16))
